# baseline (speedup 1.0000x reference)
import functools

import jax
import jax.numpy as jnp
from jax.experimental import pallas as pl
from jax.experimental.pallas import tpu as pltpu

_INV_TEMP = 10.0
_EPS_SQ = 1e-24
_PACK = 4


def _smog_logits_kernel(xr_ref, gr_ref, out_ref, xs_ref, gs_ref, *, bm, d):
    k = gs_ref.shape[0]

    @pl.when(pl.program_id(0) == 0)
    def _():
        gr = gr_ref[...]
        for p in range(_PACK):
            gs_ref[pl.Slice(p, k // _PACK, _PACK), :] = (
                gr[:, p * d:(p + 1) * d])
        g = gs_ref[...]
        gs_ref[...] = g * jax.lax.rsqrt(
            jnp.maximum(jnp.sum(g * g, axis=1, keepdims=True), _EPS_SQ))

    xr = xr_ref[...]
    for p in range(_PACK):
        xs_ref[pl.Slice(p, bm // _PACK, _PACK), :] = xr[:, p * d:(p + 1) * d]
    x = xs_ref[...]
    xs = x * (_INV_TEMP * jax.lax.rsqrt(
        jnp.maximum(jnp.sum(x * x, axis=1, keepdims=True), _EPS_SQ)))
    out_ref[...] = jax.lax.dot_general(
        xs, gs_ref[...], (((1,), (1,)), ((), ())),
        preferred_element_type=jnp.float32)


@functools.partial(jax.jit, static_argnames=("bm",))
def _smog_logits(x, group_features, bm):
    b, d = x.shape
    k, _ = group_features.shape
    bm = min(bm, b)
    xr = x.reshape(b // _PACK, d * _PACK)
    gr = group_features.reshape(k // _PACK, d * _PACK)
    return pl.pallas_call(
        functools.partial(_smog_logits_kernel, bm=bm, d=d),
        grid=(b // bm,),
        in_specs=[
            pl.BlockSpec((bm // _PACK, d * _PACK), lambda i: (i, 0)),
            pl.BlockSpec((k // _PACK, d * _PACK), lambda i: (0, 0)),
        ],
        out_specs=pl.BlockSpec((bm, k), lambda i: (i, 0)),
        out_shape=jax.ShapeDtypeStruct((b, k), jnp.float32),
        scratch_shapes=[
            pltpu.VMEM((bm, d), jnp.float32),
            pltpu.VMEM((k, d), jnp.float32),
        ],
        compiler_params=pltpu.CompilerParams(
            dimension_semantics=("arbitrary",)),
    )(xr, gr)


def kernel(x, group_features):
    return _smog_logits(x, group_features, bm=256)


# speedup vs baseline: 1.0454x; 1.0454x over previous
import functools

import jax
import jax.numpy as jnp
from jax.experimental import pallas as pl
from jax.experimental.pallas import tpu as pltpu

_INV_TEMP = 10.0
_EPS_SQ = 1e-24


def _smog_logits_kernel(x_ref, g_ref, out_ref):
    x = x_ref[...]
    g = g_ref[...]
    xs = x * (_INV_TEMP * jax.lax.rsqrt(
        jnp.maximum(jnp.sum(x * x, axis=1, keepdims=True), _EPS_SQ)))
    gs = g * jax.lax.rsqrt(
        jnp.maximum(jnp.sum(g * g, axis=1, keepdims=True), _EPS_SQ))
    out_ref[...] = jax.lax.dot_general(
        xs, gs, (((1,), (1,)), ((), ())),
        preferred_element_type=jnp.float32)


@functools.partial(jax.jit, static_argnames=("bm",))
def _smog_logits(x, group_features, bm):
    b, d = x.shape
    k, _ = group_features.shape
    bm = min(bm, b)
    return pl.pallas_call(
        _smog_logits_kernel,
        grid=(b // bm,),
        in_specs=[
            pl.BlockSpec((bm, d), lambda i: (i, 0)),
            pl.BlockSpec((k, d), lambda i: (0, 0)),
        ],
        out_specs=pl.BlockSpec((bm, k), lambda i: (i, 0)),
        out_shape=jax.ShapeDtypeStruct((b, k), jnp.float32),
        compiler_params=pltpu.CompilerParams(
            dimension_semantics=("arbitrary",)),
    )(x, group_features)


def kernel(x, group_features):
    return _smog_logits(x, group_features, bm=256)
